# baseline (speedup 1.0000x reference)
import functools
from itertools import combinations

import numpy as np
import jax
import jax.numpy as jnp
from jax import lax
from jax.experimental import pallas as pl
from jax.experimental.pallas import tpu as pltpu
from jax.experimental.pallas import tpu_sc as plsc

_B = 4096
_F = 26
_K = 16
_D0 = _F * _K
_BN_EPS = 1e-3
_FM_SCALE = 0.5 / float(np.sqrt(1.0 + _BN_EPS))

_NC = 2
_NS = 16
_NW = _NC * _NS
_SAMP_PER_W = _B // _NW
_NROW = _F
_IDX_PER_W = _SAMP_PER_W * _F
_XW_ROWS = 62500

_pairs = list(combinations(range(_F), 2))
_NPAIRS = len(_pairs)
_ONEHOT_SYM = np.zeros((_NPAIRS, _F * _F), dtype=np.float32)
for _p, (_c, _r) in enumerate(_pairs):
    _ONEHOT_SYM[_p, _r * _F + _c] = _FM_SCALE
    _ONEHOT_SYM[_p, _c * _F + _r] = _FM_SCALE
_EYE_K = np.eye(_K, dtype=np.float32)


def _sc_gather_body(idx_in, idx16_in, xvcm_hbm, xw2_hbm, xv_out, l_out,
                    idx_v, idx16_v, kidx_v, buf_v, rows_v, l_v, sem_v, sem_w):
    wid = lax.axis_index("s") * _NC + lax.axis_index("c")
    pltpu.sync_copy(idx_in.at[wid], idx_v)
    pltpu.sync_copy(idx16_in.at[wid], idx16_v)

    @pl.loop(0, _NROW)
    def _gatherw(j):
        sl = pl.ds(j * 128, 128)
        pltpu.async_copy(xw2_hbm.at[idx16_v.at[sl]], rows_v.at[sl], sem_w)

    pltpu.make_async_copy(xw2_hbm.at[pl.ds(0, _IDX_PER_W)], rows_v, sem_w).wait()

    lanes16 = lax.iota(jnp.int32, 16)

    @pl.loop(0, _SAMP_PER_W // 16)
    def _linear(g):
        base = g * (16 * _F)

        def body(f, acc):
            pos16 = base + lanes16 * _F + f
            orig16 = plsc.load_gather(idx_v, [pos16])
            lane16 = lax.bitwise_and(orig16, 15)
            val16 = plsc.load_gather(rows_v, [pos16, lane16])
            return acc + val16

        acc = pl.loop(0, _F, init_carry=jnp.zeros((16,), jnp.float32))(body)
        l_v[pl.ds(g * 16, 16)] = acc

    @pl.loop(0, _NROW)
    def _chunk(c):
        @pl.loop(0, _K)
        def _fire(k):
            @pl.loop(0, 8)
            def _build(g):
                sl16 = pl.ds(c * 128 + g * 16, 16)
                row = kidx_v.at[k]
                row[pl.ds(g * 16, 16)] = idx16_v[sl16] + k * _XW_ROWS

            pltpu.async_copy(xvcm_hbm.at[kidx_v.at[k]], buf_v.at[k], sem_v)

        @pl.loop(0, _K)
        def _drain(k):
            pltpu.make_async_copy(xvcm_hbm.at[pl.ds(0, 128)],
                                  buf_v.at[k], sem_v).wait()

        @pl.loop(0, 8)
        def _select(g):
            p16 = c * 128 + g * 16 + lanes16
            orig16 = plsc.load_gather(idx_v, [p16])
            lane16 = lax.bitwise_and(orig16, 15)
            lp16 = g * 16 + lanes16
            for k in range(_K):
                k16 = jnp.full((16,), k, jnp.int32)
                val16 = plsc.load_gather(buf_v, [k16, lp16, lane16])
                plsc.store_scatter(rows_v, [p16, k16], val16)

    pltpu.sync_copy(rows_v, xv_out.at[wid])
    pltpu.sync_copy(l_v, l_out.at[wid])


def _sc_gather(idx2d, idx16_2d, xv_cm, xw2_table):
    mesh = plsc.VectorSubcoreMesh(core_axis_name="c", subcore_axis_name="s")
    fn = pl.kernel(
        _sc_gather_body,
        out_type=[
            jax.ShapeDtypeStruct((_NW, _IDX_PER_W, _K), jnp.float32),
            jax.ShapeDtypeStruct((_NW, _SAMP_PER_W), jnp.float32),
        ],
        mesh=mesh,
        scratch_types=[
            pltpu.VMEM((_IDX_PER_W,), jnp.int32),
            pltpu.VMEM((_IDX_PER_W,), jnp.int32),
            pltpu.VMEM((_K, 128), jnp.int32),
            pltpu.VMEM((_K, 128, _K), jnp.float32),
            pltpu.VMEM((_IDX_PER_W, _K), jnp.float32),
            pltpu.VMEM((_SAMP_PER_W,), jnp.float32),
            pltpu.SemaphoreType.DMA,
            pltpu.SemaphoreType.DMA,
        ],
        compiler_params=pltpu.CompilerParams(
            use_tc_tiling_on_sc=False, needs_layout_passes=False),
    )
    return fn(idx2d, idx16_2d, xv_cm, xw2_table)


_BLK = 512


def _tc_body(xv_ref, l_ref, w0_ref, b0_ref, w1_ref, b1_ref, w2_ref, b2_ref,
             m_ref, logit_ref, sig_ref):
    x = xv_ref[...]
    h = jnp.dot(x, w0_ref[...], preferred_element_type=jnp.float32)
    h = jnp.maximum(h + b0_ref[...], 0.0)
    h = jnp.dot(h, w1_ref[...], preferred_element_type=jnp.float32)
    h = jnp.maximum(h + b1_ref[...], 0.0)
    hv = jnp.dot(h, w2_ref[...], preferred_element_type=jnp.float32)
    y = jnp.dot(x, m_ref[...], preferred_element_type=jnp.float32)
    fm = jnp.sum(x * y, axis=1, keepdims=True)
    logit = l_ref[...] + fm + hv + b2_ref[...]
    logit_ref[...] = logit
    sig_ref[...] = jax.nn.sigmoid(logit)


def _tc_dense(xv_flat, l2d, W0, b0, W1, b1, W2, b2, M):
    nblk = _B // _BLK
    full = lambda s: pl.BlockSpec(s, lambda i: (0, 0))
    return pl.pallas_call(
        _tc_body,
        grid=(nblk,),
        in_specs=[
            pl.BlockSpec((_BLK, _D0), lambda i: (i, 0)),
            pl.BlockSpec((_BLK, 1), lambda i: (i, 0)),
            full(W0.shape), full((1, b0.shape[1])),
            full(W1.shape), full((1, b1.shape[1])),
            full(W2.shape), full((1, 1)),
            full(M.shape),
        ],
        out_specs=[
            pl.BlockSpec((_BLK, 1), lambda i: (i, 0)),
            pl.BlockSpec((_BLK, 1), lambda i: (i, 0)),
        ],
        out_shape=[
            jax.ShapeDtypeStruct((_B, 1), jnp.float32),
            jax.ShapeDtypeStruct((_B, 1), jnp.float32),
        ],
        compiler_params=pltpu.CompilerParams(
            dimension_semantics=("arbitrary",),
        ),
    )(xv_flat, l2d, W0, b0, W1, b1, W2, b2, M)


def kernel(inputs, xw_table, xv_table, W0, b0, W1, b1, W2, b2, edge_weights):
    idx = inputs.astype(jnp.int32)
    idx2d = idx.reshape(_NW, _IDX_PER_W)
    idx16_2d = lax.shift_right_logical(idx2d, 4)
    xw2_table = xw_table.reshape(_XW_ROWS, _K)

    xv_cm = jnp.swapaxes(xv_table, 0, 1).reshape(1000000, _K)

    xv_g, l_g = _sc_gather(idx2d, idx16_2d, xv_cm, xw2_table)
    xv_flat = xv_g.reshape(_B, _D0)
    l2d = l_g.reshape(_B, 1)

    S = (edge_weights @ _ONEHOT_SYM).reshape(_F, _F)
    M = (S[:, None, :, None] * _EYE_K[None, :, None, :]).reshape(_D0, _D0)

    logit2, sig2 = _tc_dense(
        xv_flat, l2d, W0, b0.reshape(1, -1), W1, b1.reshape(1, -1),
        W2, b2.reshape(1, 1), M)
    return logit2.reshape(_B), sig2.reshape(_B)


# speedup vs baseline: 2.7169x; 2.7169x over previous
import functools
from itertools import combinations

import numpy as np
import jax
import jax.numpy as jnp
from jax import lax
from jax.experimental import pallas as pl
from jax.experimental.pallas import tpu as pltpu
from jax.experimental.pallas import tpu_sc as plsc

_B = 4096
_F = 26
_K = 16
_D0 = _F * _K
_BN_EPS = 1e-3
_FM_SCALE = 0.5 / float(np.sqrt(1.0 + _BN_EPS))

_NC = 2
_NS = 16
_NW = _NC * _NS
_SAMP_PER_W = _B // _NW
_NROW = _F
_IDX_PER_W = _SAMP_PER_W * _F
_XW_ROWS = 62500
_VROWS = 125000

_pairs = list(combinations(range(_F), 2))
_NPAIRS = len(_pairs)
_ONEHOT_SYM = np.zeros((_NPAIRS, _F * _F), dtype=np.float32)
for _p, (_c, _r) in enumerate(_pairs):
    _ONEHOT_SYM[_p, _r * _F + _c] = _FM_SCALE
    _ONEHOT_SYM[_p, _c * _F + _r] = _FM_SCALE
_EYE_K = np.eye(_K, dtype=np.float32)


def _sc_gather_body(idx_in, idx16_in, idx8_in, xv8_hbm, xw2_hbm, xv_out, l_out,
                    idx_v, idx16_v, idx8_v, buf_v, rows_v, l_v,
                    sem_v, sem_v2, sem_w):
    wid = lax.axis_index("s") * _NC + lax.axis_index("c")
    pltpu.sync_copy(idx_in.at[wid], idx_v)
    pltpu.sync_copy(idx16_in.at[wid], idx16_v)
    pltpu.sync_copy(idx8_in.at[wid], idx8_v)

    @pl.loop(0, _NROW)
    def _gatherw(j):
        sl = pl.ds(j * 128, 128)
        pltpu.async_copy(xw2_hbm.at[idx16_v.at[sl]], rows_v.at[sl], sem_w)

    pltpu.make_async_copy(xw2_hbm.at[pl.ds(0, _IDX_PER_W)], rows_v, sem_w).wait()

    lanes16 = lax.iota(jnp.int32, 16)

    @pl.loop(0, _SAMP_PER_W // 16)
    def _linear(g):
        base = g * (16 * _F)

        def body(f, acc):
            pos16 = base + lanes16 * _F + f
            orig16 = plsc.load_gather(idx_v, [pos16])
            lane16 = lax.bitwise_and(orig16, 15)
            val16 = plsc.load_gather(rows_v, [pos16, lane16])
            return acc + val16

        acc = pl.loop(0, _F, init_carry=jnp.zeros((16,), jnp.float32))(body)
        l_v[pl.ds(g * 16, 16)] = acc

    sems = [sem_v, sem_v2]

    def _fire(c, par):
        csl = pl.ds(c * 128, 128)
        pltpu.async_copy(xv8_hbm.at[idx8_v.at[csl]], buf_v.at[par], sems[par])

    def _drain(par):
        pltpu.make_async_copy(xv8_hbm.at[pl.ds(0, 128)], buf_v.at[par],
                              sems[par]).wait()

    _fire(0, 0)
    for c in range(_NROW):
        par = c % 2
        if c < _NROW - 1:
            _fire(c + 1, 1 - par)
        _drain(par)
        buf = buf_v.at[par]

        @pl.loop(0, 8)
        def _select(g, c=c, buf=buf):
            p16 = c * 128 + g * 16 + lanes16
            orig16 = plsc.load_gather(idx_v, [p16])
            base16 = lax.bitwise_and(orig16, 7) * 16
            lp16 = g * 16 + lanes16
            for k in range(_K):
                k16 = jnp.full((16,), k, jnp.int32)
                val16 = plsc.load_gather(buf, [lp16, base16 + k])
                plsc.store_scatter(rows_v, [p16, k16], val16)

    pltpu.sync_copy(rows_v, xv_out.at[wid])
    pltpu.sync_copy(l_v, l_out.at[wid])


def _sc_gather(idx2d, idx16_2d, idx8_2d, xv8, xw2_table):
    mesh = plsc.VectorSubcoreMesh(core_axis_name="c", subcore_axis_name="s")
    fn = pl.kernel(
        _sc_gather_body,
        out_type=[
            jax.ShapeDtypeStruct((_NW, _IDX_PER_W, _K), jnp.float32),
            jax.ShapeDtypeStruct((_NW, _SAMP_PER_W), jnp.float32),
        ],
        mesh=mesh,
        scratch_types=[
            pltpu.VMEM((_IDX_PER_W,), jnp.int32),
            pltpu.VMEM((_IDX_PER_W,), jnp.int32),
            pltpu.VMEM((_IDX_PER_W,), jnp.int32),
            pltpu.VMEM((2, 128, 128), jnp.float32),
            pltpu.VMEM((_IDX_PER_W, _K), jnp.float32),
            pltpu.VMEM((_SAMP_PER_W,), jnp.float32),
            pltpu.SemaphoreType.DMA,
            pltpu.SemaphoreType.DMA,
            pltpu.SemaphoreType.DMA,
        ],
        compiler_params=pltpu.CompilerParams(
            use_tc_tiling_on_sc=False, needs_layout_passes=False),
    )
    return fn(idx2d, idx16_2d, idx8_2d, xv8, xw2_table)


_BLK = 512


def _tc_body(xv_ref, l_ref, w0_ref, b0_ref, w1_ref, b1_ref, w2_ref, b2_ref,
             m_ref, logit_ref, sig_ref):
    x = xv_ref[...]
    h = jnp.dot(x, w0_ref[...], preferred_element_type=jnp.float32)
    h = jnp.maximum(h + b0_ref[...], 0.0)
    h = jnp.dot(h, w1_ref[...], preferred_element_type=jnp.float32)
    h = jnp.maximum(h + b1_ref[...], 0.0)
    hv = jnp.dot(h, w2_ref[...], preferred_element_type=jnp.float32)
    y = jnp.dot(x, m_ref[...], preferred_element_type=jnp.float32)
    fm = jnp.sum(x * y, axis=1, keepdims=True)
    logit = l_ref[...] + fm + hv + b2_ref[...]
    logit_ref[...] = logit
    sig_ref[...] = jax.nn.sigmoid(logit)


def _tc_dense(xv_flat, l2d, W0, b0, W1, b1, W2, b2, M):
    nblk = _B // _BLK
    full = lambda s: pl.BlockSpec(s, lambda i: (0, 0))
    return pl.pallas_call(
        _tc_body,
        grid=(nblk,),
        in_specs=[
            pl.BlockSpec((_BLK, _D0), lambda i: (i, 0)),
            pl.BlockSpec((_BLK, 1), lambda i: (i, 0)),
            full(W0.shape), full((1, b0.shape[1])),
            full(W1.shape), full((1, b1.shape[1])),
            full(W2.shape), full((1, 1)),
            full(M.shape),
        ],
        out_specs=[
            pl.BlockSpec((_BLK, 1), lambda i: (i, 0)),
            pl.BlockSpec((_BLK, 1), lambda i: (i, 0)),
        ],
        out_shape=[
            jax.ShapeDtypeStruct((_B, 1), jnp.float32),
            jax.ShapeDtypeStruct((_B, 1), jnp.float32),
        ],
        compiler_params=pltpu.CompilerParams(
            dimension_semantics=("arbitrary",),
        ),
    )(xv_flat, l2d, W0, b0, W1, b1, W2, b2, M)


def kernel(inputs, xw_table, xv_table, W0, b0, W1, b1, W2, b2, edge_weights):
    idx = inputs.astype(jnp.int32)
    idx2d = idx.reshape(_NW, _IDX_PER_W)
    idx16_2d = lax.shift_right_logical(idx2d, 4)
    idx8_2d = lax.shift_right_logical(idx2d, 3)
    xw2_table = xw_table.reshape(_XW_ROWS, _K)

    xv8 = xv_table.reshape(_VROWS, 128)

    xv_g, l_g = _sc_gather(idx2d, idx16_2d, idx8_2d, xv8, xw2_table)
    xv_flat = xv_g.reshape(_B, _D0)
    l2d = l_g.reshape(_B, 1)

    S = (edge_weights @ _ONEHOT_SYM).reshape(_F, _F)
    M = (S[:, None, :, None] * _EYE_K[None, :, None, :]).reshape(_D0, _D0)

    logit2, sig2 = _tc_dense(
        xv_flat, l2d, W0, b0.reshape(1, -1), W1, b1.reshape(1, -1),
        W2, b2.reshape(1, 1), M)
    return logit2.reshape(_B), sig2.reshape(_B)


# speedup vs baseline: 2.8981x; 1.0667x over previous
import functools
from itertools import combinations

import numpy as np
import jax
import jax.numpy as jnp
from jax import lax
from jax.experimental import pallas as pl
from jax.experimental.pallas import tpu as pltpu
from jax.experimental.pallas import tpu_sc as plsc

_B = 4096
_F = 26
_K = 16
_D0 = _F * _K
_BN_EPS = 1e-3
_FM_SCALE = 0.5 / float(np.sqrt(1.0 + _BN_EPS))

_NC = 2
_NS = 16
_NW = _NC * _NS
_SAMP_PER_W = _B // _NW
_NROW = _F
_IDX_PER_W = _SAMP_PER_W * _F
_XW_ROWS = 62500
_VROWS = 125000

_pairs = list(combinations(range(_F), 2))
_NPAIRS = len(_pairs)
_ONEHOT_SYM = np.zeros((_NPAIRS, _F * _F), dtype=np.float32)
for _p, (_c, _r) in enumerate(_pairs):
    _ONEHOT_SYM[_p, _r * _F + _c] = _FM_SCALE
    _ONEHOT_SYM[_p, _c * _F + _r] = _FM_SCALE
_EYE_K = np.eye(_K, dtype=np.float32)


def _sc_gather_body(idx_in, idx16_in, xv_hbm, xw2_hbm, xv_out, l_out,
                    idx_v, idx16_v, rows_v, l_v, sem_v, sem_w):
    wid = lax.axis_index("s") * _NC + lax.axis_index("c")
    pltpu.sync_copy(idx_in.at[wid], idx_v)
    pltpu.sync_copy(idx16_in.at[wid], idx16_v)

    @pl.loop(0, _NROW)
    def _gatherw(j):
        sl = pl.ds(j * 128, 128)
        pltpu.async_copy(xw2_hbm.at[idx16_v.at[sl]], rows_v.at[sl], sem_w)

    pltpu.make_async_copy(xw2_hbm.at[pl.ds(0, _IDX_PER_W)], rows_v, sem_w).wait()

    lanes16 = lax.iota(jnp.int32, 16)

    @pl.loop(0, _SAMP_PER_W // 16)
    def _linear(g):
        base = g * (16 * _F)

        def body(f, acc):
            pos16 = base + lanes16 * _F + f
            orig16 = plsc.load_gather(idx_v, [pos16])
            lane16 = lax.bitwise_and(orig16, 15)
            val16 = plsc.load_gather(rows_v, [pos16, lane16])
            return acc + val16

        acc = pl.loop(0, _F, init_carry=jnp.zeros((16,), jnp.float32))(body)
        l_v[pl.ds(g * 16, 16)] = acc

    @pl.loop(0, _NROW)
    def _gatherv(j):
        sl = pl.ds(j * 128, 128)
        pltpu.async_copy(xv_hbm.at[idx_v.at[sl]], rows_v.at[sl], sem_v)

    pltpu.make_async_copy(xv_hbm.at[pl.ds(0, _IDX_PER_W)], rows_v, sem_v).wait()

    pltpu.sync_copy(rows_v, xv_out.at[wid])
    pltpu.sync_copy(l_v, l_out.at[wid])


def _sc_gather(idx2d, idx16_2d, xv_table, xw2_table):
    mesh = plsc.VectorSubcoreMesh(core_axis_name="c", subcore_axis_name="s")
    fn = pl.kernel(
        _sc_gather_body,
        out_type=[
            jax.ShapeDtypeStruct((_NW, _IDX_PER_W, _K), jnp.float32),
            jax.ShapeDtypeStruct((_NW, _SAMP_PER_W), jnp.float32),
        ],
        mesh=mesh,
        scratch_types=[
            pltpu.VMEM((_IDX_PER_W,), jnp.int32),
            pltpu.VMEM((_IDX_PER_W,), jnp.int32),
            pltpu.VMEM((_IDX_PER_W, _K), jnp.float32),
            pltpu.VMEM((_SAMP_PER_W,), jnp.float32),
            pltpu.SemaphoreType.DMA,
            pltpu.SemaphoreType.DMA,
        ],
        compiler_params=pltpu.CompilerParams(
            use_tc_tiling_on_sc=False, needs_layout_passes=False),
    )
    return fn(idx2d, idx16_2d, xv_table, xw2_table)


_BLK = 512


def _tc_body(xv_ref, l_ref, w0_ref, b0_ref, w1_ref, b1_ref, w2_ref, b2_ref,
             m_ref, logit_ref, sig_ref):
    x = xv_ref[...]
    h = jnp.dot(x, w0_ref[...], preferred_element_type=jnp.float32)
    h = jnp.maximum(h + b0_ref[...], 0.0)
    h = jnp.dot(h, w1_ref[...], preferred_element_type=jnp.float32)
    h = jnp.maximum(h + b1_ref[...], 0.0)
    hv = jnp.dot(h, w2_ref[...], preferred_element_type=jnp.float32)
    y = jnp.dot(x, m_ref[...], preferred_element_type=jnp.float32)
    fm = jnp.sum(x * y, axis=1, keepdims=True)
    logit = l_ref[...] + fm + hv + b2_ref[...]
    logit_ref[...] = logit
    sig_ref[...] = jax.nn.sigmoid(logit)


def _tc_dense(xv_flat, l2d, W0, b0, W1, b1, W2, b2, M):
    nblk = _B // _BLK
    full = lambda s: pl.BlockSpec(s, lambda i: (0, 0))
    return pl.pallas_call(
        _tc_body,
        grid=(nblk,),
        in_specs=[
            pl.BlockSpec((_BLK, _D0), lambda i: (i, 0)),
            pl.BlockSpec((_BLK, 1), lambda i: (i, 0)),
            full(W0.shape), full((1, b0.shape[1])),
            full(W1.shape), full((1, b1.shape[1])),
            full(W2.shape), full((1, 1)),
            full(M.shape),
        ],
        out_specs=[
            pl.BlockSpec((_BLK, 1), lambda i: (i, 0)),
            pl.BlockSpec((_BLK, 1), lambda i: (i, 0)),
        ],
        out_shape=[
            jax.ShapeDtypeStruct((_B, 1), jnp.float32),
            jax.ShapeDtypeStruct((_B, 1), jnp.float32),
        ],
        compiler_params=pltpu.CompilerParams(
            dimension_semantics=("arbitrary",),
        ),
    )(xv_flat, l2d, W0, b0, W1, b1, W2, b2, M)


def kernel(inputs, xw_table, xv_table, W0, b0, W1, b1, W2, b2, edge_weights):
    idx = inputs.astype(jnp.int32)
    idx2d = idx.reshape(_NW, _IDX_PER_W)
    idx16_2d = lax.shift_right_logical(idx2d, 4)
    xw2_table = xw_table.reshape(_XW_ROWS, _K)

    xv_g, l_g = _sc_gather(idx2d, idx16_2d, xv_table, xw2_table)
    xv_flat = xv_g.reshape(_B, _D0)
    l2d = l_g.reshape(_B, 1)

    S = (edge_weights @ _ONEHOT_SYM).reshape(_F, _F)
    M = (S[:, None, :, None] * _EYE_K[None, :, None, :]).reshape(_D0, _D0)

    logit2, sig2 = _tc_dense(
        xv_flat, l2d, W0, b0.reshape(1, -1), W1, b1.reshape(1, -1),
        W2, b2.reshape(1, 1), M)
    return logit2.reshape(_B), sig2.reshape(_B)


# speedup vs baseline: 2.9172x; 1.0066x over previous
from itertools import combinations

import numpy as np
import jax
import jax.numpy as jnp
from jax import lax
from jax.experimental import pallas as pl
from jax.experimental.pallas import tpu as pltpu
from jax.experimental.pallas import tpu_sc as plsc

_B = 4096
_F = 26
_K = 16
_D0 = _F * _K
_BN_EPS = 1e-3
_FM_SCALE = 0.5 / float(np.sqrt(1.0 + _BN_EPS))

_NC = 2
_NS = 16
_NW = _NC * _NS
_SAMP_PER_W = _B // _NW
_NROW = _F
_IDX_PER_W = _SAMP_PER_W * _F
_XW_ROWS = 62500

_pairs = list(combinations(range(_F), 2))
_NPAIRS = len(_pairs)
_ONEHOT_SYM = np.zeros((_NPAIRS, _F * _F), dtype=np.float32)
for _p, (_c, _r) in enumerate(_pairs):
    _ONEHOT_SYM[_p, _r * _F + _c] = _FM_SCALE
    _ONEHOT_SYM[_p, _c * _F + _r] = _FM_SCALE
_EYE_K = np.eye(_K, dtype=np.float32)


def _sc_gather_body(idx_in, idx16_in, xv_hbm, xw2_hbm, xv_out, l_out,
                    idx_v, idx16_v, rows_v, l_v, sem_v, sem_w):
    wid = lax.axis_index("s") * _NC + lax.axis_index("c")
    pltpu.sync_copy(idx_in.at[wid], idx_v)
    pltpu.sync_copy(idx16_in.at[wid], idx16_v)

    @pl.loop(0, _NROW)
    def _gatherw(j):
        sl = pl.ds(j * 128, 128)
        pltpu.async_copy(xw2_hbm.at[idx16_v.at[sl]], rows_v.at[sl], sem_w)

    pltpu.make_async_copy(xw2_hbm.at[pl.ds(0, _IDX_PER_W)], rows_v, sem_w).wait()

    lanes16 = lax.iota(jnp.int32, 16)

    @pl.loop(0, _SAMP_PER_W // 16)
    def _linear(g):
        base = g * (16 * _F)

        def body(f, acc):
            pos16 = base + lanes16 * _F + f
            orig16 = plsc.load_gather(idx_v, [pos16])
            lane16 = lax.bitwise_and(orig16, 15)
            val16 = plsc.load_gather(rows_v, [pos16, lane16])
            return acc + val16

        acc = pl.loop(0, _F, init_carry=jnp.zeros((16,), jnp.float32))(body)
        l_v[pl.ds(g * 16, 16)] = acc

    @pl.loop(0, _NROW)
    def _gatherv(j):
        sl = pl.ds(j * 128, 128)
        pltpu.async_copy(xv_hbm.at[idx_v.at[sl]], rows_v.at[sl], sem_v)

    pltpu.make_async_copy(xv_hbm.at[pl.ds(0, _IDX_PER_W)], rows_v, sem_v).wait()

    pltpu.sync_copy(rows_v, xv_out.at[wid])
    pltpu.sync_copy(l_v, l_out.at[wid])


def _sc_gather(idx2d, idx16_2d, xv_table, xw2_table):
    mesh = plsc.VectorSubcoreMesh(core_axis_name="c", subcore_axis_name="s")
    fn = pl.kernel(
        _sc_gather_body,
        out_type=[
            jax.ShapeDtypeStruct((_NW, _IDX_PER_W, _K), jnp.float32),
            jax.ShapeDtypeStruct((_NW, _SAMP_PER_W), jnp.float32),
        ],
        mesh=mesh,
        scratch_types=[
            pltpu.VMEM((_IDX_PER_W,), jnp.int32),
            pltpu.VMEM((_IDX_PER_W,), jnp.int32),
            pltpu.VMEM((_IDX_PER_W, _K), jnp.float32),
            pltpu.VMEM((_SAMP_PER_W,), jnp.float32),
            pltpu.SemaphoreType.DMA,
            pltpu.SemaphoreType.DMA,
        ],
        compiler_params=pltpu.CompilerParams(
            use_tc_tiling_on_sc=False, needs_layout_passes=False),
    )
    return fn(idx2d, idx16_2d, xv_table, xw2_table)


_BLK = 512


def _tc_body(xv_ref, l_ref, w0_ref, b0_ref, w1_ref, b1_ref, w2_ref, b2_ref,
             m_ref, logit_ref, sig_ref):
    x = xv_ref[...]
    h = jnp.dot(x, w0_ref[...], preferred_element_type=jnp.float32)
    h = jnp.maximum(h + b0_ref[...], 0.0)
    h = jnp.dot(h, w1_ref[...], preferred_element_type=jnp.float32)
    h = jnp.maximum(h + b1_ref[...], 0.0)
    hv = jnp.dot(h, w2_ref[...], preferred_element_type=jnp.float32)
    y = jnp.dot(x, m_ref[...], preferred_element_type=jnp.float32)
    fm = jnp.sum(x * y, axis=1, keepdims=True)
    logit = l_ref[...] + fm + hv + b2_ref[...]
    logit_ref[...] = logit
    sig_ref[...] = jax.nn.sigmoid(logit)


def _tc_dense(xv_flat, l2d, W0, b0, W1, b1, W2, b2, M):
    nblk = _B // _BLK
    full = lambda s: pl.BlockSpec(s, lambda i: (0, 0))
    return pl.pallas_call(
        _tc_body,
        grid=(nblk,),
        in_specs=[
            pl.BlockSpec((_BLK, _D0), lambda i: (i, 0)),
            pl.BlockSpec((_BLK, 1), lambda i: (i, 0)),
            full(W0.shape), full((1, b0.shape[1])),
            full(W1.shape), full((1, b1.shape[1])),
            full(W2.shape), full((1, 1)),
            full(M.shape),
        ],
        out_specs=[
            pl.BlockSpec((_BLK, 1), lambda i: (i, 0)),
            pl.BlockSpec((_BLK, 1), lambda i: (i, 0)),
        ],
        out_shape=[
            jax.ShapeDtypeStruct((_B, 1), jnp.float32),
            jax.ShapeDtypeStruct((_B, 1), jnp.float32),
        ],
        compiler_params=pltpu.CompilerParams(
            dimension_semantics=("arbitrary",),
        ),
    )(xv_flat, l2d, W0, b0, W1, b1, W2, b2, M)


def kernel(inputs, xw_table, xv_table, W0, b0, W1, b1, W2, b2, edge_weights):
    idx = inputs.astype(jnp.int32)
    idx2d = idx.reshape(_NW, _IDX_PER_W)
    idx16_2d = lax.shift_right_logical(idx2d, 4)
    xw2_table = xw_table.reshape(_XW_ROWS, _K)

    xv_g, l_g = _sc_gather(idx2d, idx16_2d, xv_table, xw2_table)
    xv_flat = xv_g.reshape(_B, _D0)
    l2d = l_g.reshape(_B, 1)

    S = (edge_weights @ _ONEHOT_SYM).reshape(_F, _F)
    M = (S[:, None, :, None] * _EYE_K[None, :, None, :]).reshape(_D0, _D0)

    logit2, sig2 = _tc_dense(
        xv_flat, l2d, W0, b0.reshape(1, -1), W1, b1.reshape(1, -1),
        W2, b2.reshape(1, 1), M)
    return logit2.reshape(_B), sig2.reshape(_B)
